# baseline (speedup 1.0000x reference)
import jax
import jax.numpy as jnp
from jax import lax
from jax.experimental import pallas as pl
from jax.experimental.pallas import tpu as pltpu
from jax.experimental.pallas import tpu_sc as plsc

NUM_CORES = 2
NUM_SUBCORES = 16
NUM_WORKERS = NUM_CORES * NUM_SUBCORES
LANES = 16

BATCH = 16384
NFEATURE = 32
B_PER_W = BATCH // NUM_WORKERS
CHUNK = 128
NCHUNKS = B_PER_W // CHUNK


def _make_lookup():
  mesh = plsc.VectorSubcoreMesh(
      core_axis_name="c", subcore_axis_name="s")

  @pl.kernel(
      out_type=jax.ShapeDtypeStruct((BATCH, NFEATURE), jnp.float32),
      mesh=mesh,
      scratch_types=[
          pltpu.VMEM((B_PER_W,), jnp.int32),
          pltpu.VMEM((B_PER_W, NFEATURE), jnp.float32),
          pltpu.SemaphoreType.DMA,
      ],
      compiler_params=pltpu.CompilerParams(use_tc_tiling_on_sc=False),
  )
  def lookup(ids_hbm, table_hbm, out_hbm, idx_v, rows_v, sem):
    wid = lax.axis_index("s") * NUM_CORES + lax.axis_index("c")
    base = wid * B_PER_W

    pltpu.sync_copy(ids_hbm.at[pl.ds(base, B_PER_W)], idx_v)

    for i in range(B_PER_W // LANES):
      sl = pl.ds(i * LANES, LANES)
      idx_v[sl] = idx_v[sl] + 1

    copies = []
    for j in range(NCHUNKS):
      sl = pl.ds(j * CHUNK, CHUNK)
      copies.append(
          pltpu.async_copy(table_hbm.at[idx_v.at[sl]], rows_v.at[sl], sem))
    for c in copies:
      c.wait()

    pltpu.sync_copy(rows_v, out_hbm.at[pl.ds(base, B_PER_W)])

  return lookup


_lookup = _make_lookup()


def kernel(user_id, table):
  return _lookup(user_id.astype(jnp.int32), table)


# speedup vs baseline: 3.0519x; 3.0519x over previous
import jax
import jax.numpy as jnp
from jax import lax
from jax.experimental import pallas as pl
from jax.experimental.pallas import tpu as pltpu
from jax.experimental.pallas import tpu_sc as plsc

NUM_CORES = 2
NUM_SUBCORES = 16
NUM_WORKERS = NUM_CORES * NUM_SUBCORES
LANES = 16

BATCH = 16384
NFEATURE = 32
B_PER_W = BATCH // NUM_WORKERS
NBUF = 8


def _make_lookup():
  mesh = plsc.VectorSubcoreMesh(
      core_axis_name="c", subcore_axis_name="s")

  @pl.kernel(
      out_type=jax.ShapeDtypeStruct((BATCH, NFEATURE), jnp.float32),
      mesh=mesh,
      scratch_types=[
          pltpu.VMEM((B_PER_W,), jnp.int32),
          pltpu.VMEM((NBUF, NFEATURE, 128), jnp.float32),
          pltpu.VMEM((B_PER_W, NFEATURE), jnp.float32),
          pltpu.SemaphoreType.DMA,
      ],
      compiler_params=pltpu.CompilerParams(
          needs_layout_passes=False, disable_bounds_checks=True),
  )
  def lookup(ids_hbm, tablet_hbm, out_hbm, idx_v, blk_v, stag_v, sem):
    wid = lax.axis_index("s") * NUM_CORES + lax.axis_index("c")
    base = wid * B_PER_W

    pltpu.sync_copy(ids_hbm.at[pl.ds(base, B_PER_W)], idx_v)

    rows = lax.iota(jnp.int32, LANES)

    def half(ci, v, lo_lane):
      copies = []
      for l in range(lo_lane, lo_lane + NBUF):
        r = v[l]
        j128 = pl.multiple_of((r >> 7) << 7, 128)
        copies.append(pltpu.async_copy(
            tablet_hbm.at[:, pl.ds(j128, 128)],
            blk_v.at[l - lo_lane], sem))
      for c in copies:
        c.wait()
      for l in range(lo_lane, lo_lane + NBUF):
        r = v[l]
        cols = jnp.full((LANES,), r & 127, jnp.int32)
        lo = plsc.load_gather(blk_v.at[l - lo_lane], [rows, cols])
        hi = plsc.load_gather(blk_v.at[l - lo_lane], [rows + LANES, cols])
        i = jnp.full((LANES,), ci * LANES + l, jnp.int32)
        plsc.store_scatter(stag_v, [i, rows], lo)
        plsc.store_scatter(stag_v, [i, rows + LANES], hi)

    def chunk(ci, _):
      v = idx_v[pl.ds(ci * LANES, LANES)] + 1
      half(ci, v, 0)
      half(ci, v, NBUF)
      return 0

    lax.fori_loop(0, B_PER_W // LANES, chunk, 0)

    pltpu.sync_copy(stag_v, out_hbm.at[pl.ds(base, B_PER_W), :])

  return lookup


_lookup = _make_lookup()


def kernel(user_id, table):
  return _lookup(user_id.astype(jnp.int32), table.T)
